# Initial kernel scaffold; baseline (speedup 1.0000x reference)
import jax
import jax.numpy as jnp
from jax.experimental import pallas as pl


def kernel(x, edge_index, edge_attr, Wm0, bm0, root0, bias0, gamma0, beta0, Wm1, bm1, root1, bias1, gamma1, beta1, Wm2, bm2, root2, bias2, gamma2, beta2, Wmf, bmf, rootf, biasf):
    raise NotImplementedError("write your pallas kernel here")


# speedup vs baseline: 5.7948x; 5.7948x over previous
import functools

import jax
import jax.numpy as jnp
from jax import lax
from jax.experimental import pallas as pl
from jax.experimental.pallas import tpu as pltpu
from jax.experimental.pallas import tpu_sc as plsc

N = 10000
E = 160000
D = 16
FT = 16

NC = 2
NS = 16
NW = NC * NS
EPW = E // NW
CHUNK = 125
NCH = EPW // CHUNK
IDX_ROWS = E // CHUNK
NPT = N // NS


def _mesh():
    return plsc.VectorSubcoreMesh(
        core_axis_name="c", subcore_axis_name="s", num_cores=NC, num_subcores=NS
    )


_SC_PARAMS = pltpu.CompilerParams(use_tc_tiling_on_sc=False)


@functools.cache
def _make_sc_gather():
    return functools.partial(
        pl.kernel,
        mesh=_mesh(),
        compiler_params=_SC_PARAMS,
        out_type=jax.ShapeDtypeStruct((E, D), jnp.float32),
        scratch_types=[
            pltpu.VMEM((NCH, CHUNK), jnp.int32),
            pltpu.VMEM((EPW, D), jnp.float32),
            pltpu.SemaphoreType.DMA,
        ],
    )(_sc_gather_body)


def _sc_gather_body(table_hbm, idx_hbm, out_hbm, idx_v, rows_v, sem):
    c = lax.axis_index("c")
    s = lax.axis_index("s")
    wid = c * NS + s
    pltpu.sync_copy(idx_hbm.at[pl.ds(wid * NCH, NCH)], idx_v)

    def fire(j, carry):
        pltpu.async_copy(
            table_hbm.at[idx_v.at[j]], rows_v.at[pl.ds(j * CHUNK, CHUNK)], sem
        )
        return carry

    lax.fori_loop(0, NCH, fire, 0)
    pltpu.make_async_copy(out_hbm.at[pl.ds(wid * EPW, EPW)], rows_v, sem).wait()
    pltpu.sync_copy(rows_v, out_hbm.at[pl.ds(wid * EPW, EPW)])


@functools.cache
def _make_sc_scatter():
    return functools.partial(
        pl.kernel,
        mesh=_mesh(),
        compiler_params=_SC_PARAMS,
        out_type=jax.ShapeDtypeStruct((NC * N, D), jnp.float32),
        scratch_types=[
            pltpu.VMEM_SHARED((N, D), jnp.float32),
            pltpu.VMEM((NCH, CHUNK), jnp.int32),
            pltpu.VMEM((EPW, D), jnp.float32),
        ],
    )(_sc_scatter_body)


def _sc_scatter_body(msg_hbm, idx_hbm, zeros_hbm, out_hbm, acc_sh, idx_v, data_v):
    c = lax.axis_index("c")
    s = lax.axis_index("s")
    wid = c * NS + s
    pltpu.sync_copy(zeros_hbm, acc_sh.at[pl.ds(s * NPT, NPT)])
    pltpu.sync_copy(idx_hbm.at[pl.ds(wid * NCH, NCH)], idx_v)
    pltpu.sync_copy(msg_hbm.at[pl.ds(wid * EPW, EPW)], data_v)
    plsc.subcore_barrier()

    def body(j, carry):
        pltpu.sync_copy(
            data_v.at[pl.ds(j * CHUNK, CHUNK)], acc_sh.at[idx_v.at[j]], add=True
        )
        return carry

    lax.fori_loop(0, NCH, body, 0)
    plsc.subcore_barrier()
    pltpu.sync_copy(
        acc_sh.at[pl.ds(s * NPT, NPT)], out_hbm.at[pl.ds(c * N + s * NPT, NPT)]
    )


@functools.cache
def _make_sc_count():
    return functools.partial(
        pl.kernel,
        mesh=_mesh(),
        compiler_params=_SC_PARAMS,
        out_type=jax.ShapeDtypeStruct((NC * N, D), jnp.float32),
        scratch_types=[
            pltpu.VMEM_SHARED((N, D), jnp.float32),
            pltpu.VMEM((NCH, CHUNK), jnp.int32),
            pltpu.VMEM((CHUNK, D), jnp.float32),
        ],
    )(_sc_count_body)


def _sc_count_body(idx_hbm, ones_hbm, zeros_hbm, out_hbm, acc_sh, idx_v, ones_v):
    c = lax.axis_index("c")
    s = lax.axis_index("s")
    wid = c * NS + s
    pltpu.sync_copy(zeros_hbm, acc_sh.at[pl.ds(s * NPT, NPT)])
    pltpu.sync_copy(idx_hbm.at[pl.ds(wid * NCH, NCH)], idx_v)
    pltpu.sync_copy(ones_hbm, ones_v)
    plsc.subcore_barrier()

    def body(j, carry):
        pltpu.sync_copy(ones_v, acc_sh.at[idx_v.at[j]], add=True)
        return carry

    lax.fori_loop(0, NCH, body, 0)
    plsc.subcore_barrier()
    pltpu.sync_copy(
        acc_sh.at[pl.ds(s * NPT, NPT)], out_hbm.at[pl.ds(c * N + s * NPT, NPT)]
    )


BQ = 400
ER = E * D // 128
NR = N * D // 128


def _split(a):
    hi = a.astype(jnp.bfloat16)
    lo = (a - hi.astype(jnp.float32)).astype(jnp.bfloat16)
    return hi, lo


def _bdot(a, b):
    return jnp.dot(a, b, preferred_element_type=jnp.float32)


def _dot3(x, wh, wl):
    xh, xl = _split(x)
    return _bdot(xh, wh) + (_bdot(xh, wl) + _bdot(xl, wh))


def _msg_body(ea_ref, hs_ref, wh_ref, wl_ref, bm_ref, r_ref, out_ref):
    t = jnp.tanh(_dot3(ea_ref[...], wh_ref[...], wl_ref[...]) + bm_ref[...])
    hsh, hsl = _split(hs_ref[...])
    p = t * (_bdot(hsh, r_ref[...]) + _bdot(hsl, r_ref[...]))
    m = p[:, :1024] + p[:, 1024:]
    m = m[:, :512] + m[:, 512:]
    m = m[:, :256] + m[:, 256:]
    out_ref[...] = m[:, :128] + m[:, 128:]


_msg_call = pl.pallas_call(
    _msg_body,
    grid=(ER // BQ,),
    in_specs=[
        pl.BlockSpec((BQ, 128), lambda i: (i, 0)),
        pl.BlockSpec((BQ, 128), lambda i: (i, 0)),
        pl.BlockSpec((128, 16 * 128), lambda i: (0, 0)),
        pl.BlockSpec((128, 16 * 128), lambda i: (0, 0)),
        pl.BlockSpec((1, 16 * 128), lambda i: (0, 0)),
        pl.BlockSpec((128, 16 * 128), lambda i: (0, 0)),
    ],
    out_specs=pl.BlockSpec((BQ, 128), lambda i: (i, 0)),
    out_shape=jax.ShapeDtypeStruct((ER, 128), jnp.float32),
)


def _msgf_body(ea_ref, hs_ref, wh_ref, wl_ref, bm_ref, out_ref):
    t = jnp.tanh(_dot3(ea_ref[...], wh_ref[...], wl_ref[...]) + bm_ref[...])
    out_ref[...] = t * hs_ref[...]


_msgf_call = pl.pallas_call(
    _msgf_body,
    grid=(ER // BQ,),
    in_specs=[
        pl.BlockSpec((BQ, 128), lambda i: (i, 0)),
        pl.BlockSpec((BQ, 128), lambda i: (i, 0)),
        pl.BlockSpec((128, 128), lambda i: (0, 0)),
        pl.BlockSpec((128, 128), lambda i: (0, 0)),
        pl.BlockSpec((1, 128), lambda i: (0, 0)),
    ],
    out_specs=pl.BlockSpec((BQ, 128), lambda i: (i, 0)),
    out_shape=jax.ShapeDtypeStruct((ER, 128), jnp.float32),
)


def _tile8(v16):
    return jnp.concatenate([v16] * 8, axis=1)


def _fold8(v):
    m = v[:, :64] + v[:, 64:]
    m = m[:, :32] + m[:, 32:]
    return m[:, :16] + m[:, 16:]


def _node_body(pacc_ref, cacc_ref, h_ref, rh_ref, rl_ref, bias_ref, gamma_ref,
               beta_ref, out_ref):
    pa = pacc_ref[...]
    ca = cacc_ref[...]
    h = h_ref[...]
    agg = pa[:NR] + pa[NR:]
    cnt = ca[:NR] + ca[NR:]
    inv = 1.0 / jnp.maximum(cnt, 1.0)
    t = agg * inv + _dot3(h, rh_ref[...], rl_ref[...]) + bias_ref[...]
    mu = _tile8(_fold8(jnp.mean(t, axis=0, keepdims=True)) * 0.125)
    e = t - mu
    var = _tile8(_fold8(jnp.mean(e * e, axis=0, keepdims=True)) * 0.125)
    tb = e * lax.rsqrt(var + 1e-5) * gamma_ref[...] + beta_ref[...]
    out_ref[...] = h + jnp.maximum(tb, 0.0)


_node_call = pl.pallas_call(
    _node_body,
    out_shape=jax.ShapeDtypeStruct((NR, 128), jnp.float32),
)


def _final_body(pacc_ref, cacc_ref, h_ref, rootf_ref, biasf_ref, fh_ref,
                out_ref):
    pa = pacc_ref[...]
    ca = cacc_ref[...]
    aggf = pa[:NR] + pa[NR:]
    cnt = ca[:NR] + ca[NR:]
    q = aggf / jnp.maximum(cnt, 1.0) + h_ref[...] * rootf_ref[...]
    qh, ql = _split(q)
    f = fh_ref[...]
    out_ref[...] = _bdot(qh, f) + _bdot(ql, f) + biasf_ref[...]


_final_call = pl.pallas_call(
    _final_body,
    out_shape=jax.ShapeDtypeStruct((NR, 8), jnp.float32),
)


def kernel(x, edge_index, edge_attr, Wm0, bm0, root0, bias0, gamma0, beta0,
           Wm1, bm1, root1, bias1, gamma1, beta1, Wm2, bm2, root2, bias2,
           gamma2, beta2, Wmf, bmf, rootf, biasf):
    f32 = jnp.float32
    bf16 = jnp.bfloat16
    src = edge_index[0].reshape(IDX_ROWS, CHUNK)
    dst = edge_index[1].reshape(IDX_ROWS, CHUNK)
    zeros_t = jnp.zeros((NPT, D), f32)
    ones_c = jnp.ones((CHUNK, D), f32)
    eye8 = jnp.eye(8, dtype=f32)

    def wsplit(w):
        hi = w.astype(bf16)
        lo = (w - hi.astype(f32)).astype(bf16)
        return hi, lo

    def kron_w(w):
        if w.shape[1] == D * D:
            wt = w.reshape(FT, D, D)
            k = jnp.einsum('xy,qio->xqiyo', eye8, wt).reshape(128, 16 * 128)
        else:
            k = jnp.einsum('xy,qo->xqyo', eye8, w).reshape(128, 8 * w.shape[1])
        return wsplit(k)

    def bm_cols(bm):
        return jnp.broadcast_to(bm.reshape(D, 1, D), (D, 8, D)).reshape(
            1, 16 * 128
        )

    def tile8v(v):
        return jnp.tile(v.reshape(1, D), (1, 8))

    r8 = jnp.einsum(
        'xy,ki,o->xkiyo', eye8, jnp.eye(D, dtype=f32), jnp.ones((D,), f32)
    ).reshape(128, 16 * 128).astype(bf16)
    f8h = jnp.einsum('xy,o->xoy', eye8, jnp.ones((D,), f32)).reshape(
        128, 8
    ).astype(bf16)

    sc_count = _make_sc_count()
    sc_gather = _make_sc_gather()
    sc_scatter = _make_sc_scatter()

    cacc = sc_count(dst, ones_c, zeros_t).reshape(2 * NR, 128)
    ea128 = edge_attr.reshape(ER, 128)

    h = x
    h128 = x.reshape(NR, 128)
    for Wm, bm, root, bias, gamma, beta in (
        (Wm0, bm0, root0, bias0, gamma0, beta0),
        (Wm1, bm1, root1, bias1, gamma1, beta1),
        (Wm2, bm2, root2, bias2, gamma2, beta2),
    ):
        hs = sc_gather(h, src).reshape(ER, 128)
        wh, wl = kron_w(Wm)
        msg = _msg_call(ea128, hs, wh, wl, bm_cols(bm), r8)
        pacc = sc_scatter(msg.reshape(E, D), dst, zeros_t).reshape(2 * NR, 128)
        rh, rl = kron_w(root)
        h128 = _node_call(pacc, cacc, h128, rh, rl, tile8v(bias),
                          tile8v(gamma), tile8v(beta))
        h = h128.reshape(N, D)

    hs = sc_gather(h, src).reshape(ER, 128)
    wfh, wfl = kron_w(Wmf)
    prodf = _msgf_call(ea128, hs, wfh, wfl, tile8v(bmf))
    paccf = sc_scatter(prodf.reshape(E, D), dst, zeros_t).reshape(2 * NR, 128)
    out8 = _final_call(paccf, cacc, h128, tile8v(rootf.reshape(D)),
                       biasf.reshape(1, 1), f8h)
    return out8.reshape(N, 1)


# speedup vs baseline: 6.8634x; 1.1844x over previous
import functools

import jax
import jax.numpy as jnp
from jax import lax
from jax.experimental import pallas as pl
from jax.experimental.pallas import tpu as pltpu
from jax.experimental.pallas import tpu_sc as plsc

N = 10000
E = 160000
D = 16
FT = 16

NC = 2
NS = 16
NW = NC * NS
EPW = E // NW
CHUNK = 125
NCH = EPW // CHUNK
IDX_ROWS = E // CHUNK
NPT = N // NS


def _mesh():
    return plsc.VectorSubcoreMesh(
        core_axis_name="c", subcore_axis_name="s", num_cores=NC, num_subcores=NS
    )


_SC_PARAMS = pltpu.CompilerParams(use_tc_tiling_on_sc=False)


@functools.cache
def _make_sc_gather():
    return functools.partial(
        pl.kernel,
        mesh=_mesh(),
        compiler_params=_SC_PARAMS,
        out_type=jax.ShapeDtypeStruct((E, D), jnp.float32),
        scratch_types=[
            pltpu.VMEM((NCH, CHUNK), jnp.int32),
            pltpu.VMEM((EPW, D), jnp.float32),
            pltpu.SemaphoreType.DMA,
        ],
    )(_sc_gather_body)


def _sc_gather_body(table_hbm, idx_hbm, out_hbm, idx_v, rows_v, sem):
    c = lax.axis_index("c")
    s = lax.axis_index("s")
    wid = c * NS + s
    pltpu.sync_copy(idx_hbm.at[pl.ds(wid * NCH, NCH)], idx_v)

    def fire(j, carry):
        pltpu.async_copy(
            table_hbm.at[idx_v.at[j]], rows_v.at[pl.ds(j * CHUNK, CHUNK)], sem
        )
        return carry

    lax.fori_loop(0, NCH, fire, 0)
    pltpu.make_async_copy(out_hbm.at[pl.ds(wid * EPW, EPW)], rows_v, sem).wait()
    pltpu.sync_copy(rows_v, out_hbm.at[pl.ds(wid * EPW, EPW)])


@functools.cache
def _make_sc_scatter():
    return functools.partial(
        pl.kernel,
        mesh=_mesh(),
        compiler_params=_SC_PARAMS,
        out_type=jax.ShapeDtypeStruct((NC * N, D), jnp.float32),
        scratch_types=[
            pltpu.VMEM_SHARED((N, D), jnp.float32),
            pltpu.VMEM((NCH, CHUNK), jnp.int32),
            pltpu.VMEM((EPW, D), jnp.float32),
        ],
    )(_sc_scatter_body)


def _sc_scatter_body(msg_hbm, idx_hbm, zeros_hbm, out_hbm, acc_sh, idx_v, data_v):
    c = lax.axis_index("c")
    s = lax.axis_index("s")
    wid = c * NS + s
    pltpu.sync_copy(zeros_hbm, acc_sh.at[pl.ds(s * NPT, NPT)])
    pltpu.sync_copy(idx_hbm.at[pl.ds(wid * NCH, NCH)], idx_v)
    pltpu.sync_copy(msg_hbm.at[pl.ds(wid * EPW, EPW)], data_v)
    plsc.subcore_barrier()

    def body(j, carry):
        pltpu.sync_copy(
            data_v.at[pl.ds(j * CHUNK, CHUNK)], acc_sh.at[idx_v.at[j]], add=True
        )
        return carry

    lax.fori_loop(0, NCH, body, 0)
    plsc.subcore_barrier()
    pltpu.sync_copy(
        acc_sh.at[pl.ds(s * NPT, NPT)], out_hbm.at[pl.ds(c * N + s * NPT, NPT)]
    )


@functools.cache
def _make_sc_count():
    return functools.partial(
        pl.kernel,
        mesh=_mesh(),
        compiler_params=_SC_PARAMS,
        out_type=jax.ShapeDtypeStruct((NC * N, D), jnp.float32),
        scratch_types=[
            pltpu.VMEM_SHARED((N, D), jnp.float32),
            pltpu.VMEM((NCH, CHUNK), jnp.int32),
            pltpu.VMEM((CHUNK, D), jnp.float32),
        ],
    )(_sc_count_body)


def _sc_count_body(idx_hbm, ones_hbm, zeros_hbm, out_hbm, acc_sh, idx_v, ones_v):
    c = lax.axis_index("c")
    s = lax.axis_index("s")
    wid = c * NS + s
    pltpu.sync_copy(zeros_hbm, acc_sh.at[pl.ds(s * NPT, NPT)])
    pltpu.sync_copy(idx_hbm.at[pl.ds(wid * NCH, NCH)], idx_v)
    pltpu.sync_copy(ones_hbm, ones_v)
    plsc.subcore_barrier()

    def body(j, carry):
        pltpu.sync_copy(ones_v, acc_sh.at[idx_v.at[j]], add=True)
        return carry

    lax.fori_loop(0, NCH, body, 0)
    plsc.subcore_barrier()
    pltpu.sync_copy(
        acc_sh.at[pl.ds(s * NPT, NPT)], out_hbm.at[pl.ds(c * N + s * NPT, NPT)]
    )


BQ = 400
ER = E * D // 128
NR = N * D // 128


def _split(a):
    hi = a.astype(jnp.bfloat16)
    lo = (a - hi.astype(jnp.float32)).astype(jnp.bfloat16)
    return hi, lo


def _bdot(a, b):
    return jnp.dot(a, b, preferred_element_type=jnp.float32)


def _dot3(x, wh, w2):
    xh, xl = _split(x)
    x2 = jnp.concatenate([xh, xl], axis=1)
    return _bdot(xh, wh) + _bdot(x2, w2)


def _msg_body(ea_ref, hs_ref, wh_ref, w2_ref, bm_ref, r2_ref, out_ref):
    eah, eal = _split(ea_ref[...])
    hsh, hsl = _split(hs_ref[...])
    ea2 = jnp.concatenate([eah, eal], axis=1)
    hs2 = jnp.concatenate([hsh, hsl], axis=1)
    acc = None
    for ii in range(8):
        cs = pl.ds(ii * 256, 256)
        pre = _bdot(eah, wh_ref[:, cs]) + _bdot(ea2, w2_ref[:, cs]) + bm_ref[:, cs]
        t = jnp.tanh(pre)
        p = t * _bdot(hs2, r2_ref[:, cs])
        m = p[:, :128] + p[:, 128:]
        acc = m if acc is None else acc + m
    out_ref[...] = acc


_msg_call = pl.pallas_call(
    _msg_body,
    grid=(ER // BQ,),
    in_specs=[
        pl.BlockSpec((BQ, 128), lambda i: (i, 0)),
        pl.BlockSpec((BQ, 128), lambda i: (i, 0)),
        pl.BlockSpec((128, 16 * 128), lambda i: (0, 0)),
        pl.BlockSpec((256, 16 * 128), lambda i: (0, 0)),
        pl.BlockSpec((1, 16 * 128), lambda i: (0, 0)),
        pl.BlockSpec((256, 16 * 128), lambda i: (0, 0)),
    ],
    out_specs=pl.BlockSpec((BQ, 128), lambda i: (i, 0)),
    out_shape=jax.ShapeDtypeStruct((ER, 128), jnp.float32),
)


def _msgf_body(ea_ref, hs_ref, wh_ref, w2_ref, bm_ref, out_ref):
    eah, eal = _split(ea_ref[...])
    ea2 = jnp.concatenate([eah, eal], axis=1)
    t = jnp.tanh(_bdot(eah, wh_ref[...]) + _bdot(ea2, w2_ref[...]) + bm_ref[...])
    out_ref[...] = t * hs_ref[...]


_msgf_call = pl.pallas_call(
    _msgf_body,
    grid=(ER // BQ,),
    in_specs=[
        pl.BlockSpec((BQ, 128), lambda i: (i, 0)),
        pl.BlockSpec((BQ, 128), lambda i: (i, 0)),
        pl.BlockSpec((128, 128), lambda i: (0, 0)),
        pl.BlockSpec((256, 128), lambda i: (0, 0)),
        pl.BlockSpec((1, 128), lambda i: (0, 0)),
    ],
    out_specs=pl.BlockSpec((BQ, 128), lambda i: (i, 0)),
    out_shape=jax.ShapeDtypeStruct((ER, 128), jnp.float32),
)


def _tile8(v16):
    return jnp.concatenate([v16] * 8, axis=1)


def _fold8(v):
    m = v[:, :64] + v[:, 64:]
    m = m[:, :32] + m[:, 32:]
    return m[:, :16] + m[:, 16:]


def _node_body(pacc_ref, cacc_ref, h_ref, rh_ref, r2_ref, bias_ref, gamma_ref,
               beta_ref, out_ref):
    pa = pacc_ref[...]
    ca = cacc_ref[...]
    h = h_ref[...]
    agg = pa[:NR] + pa[NR:]
    cnt = ca[:NR] + ca[NR:]
    inv = 1.0 / jnp.maximum(cnt, 1.0)
    t = agg * inv + _dot3(h, rh_ref[...], r2_ref[...]) + bias_ref[...]
    mu = _tile8(_fold8(jnp.mean(t, axis=0, keepdims=True)) * 0.125)
    e = t - mu
    var = _tile8(_fold8(jnp.mean(e * e, axis=0, keepdims=True)) * 0.125)
    tb = e * lax.rsqrt(var + 1e-5) * gamma_ref[...] + beta_ref[...]
    out_ref[...] = h + jnp.maximum(tb, 0.0)


_node_call = pl.pallas_call(
    _node_body,
    out_shape=jax.ShapeDtypeStruct((NR, 128), jnp.float32),
)


def _final_body(pacc_ref, cacc_ref, h_ref, rootf_ref, biasf_ref, fh_ref,
                out_ref):
    pa = pacc_ref[...]
    ca = cacc_ref[...]
    aggf = pa[:NR] + pa[NR:]
    cnt = ca[:NR] + ca[NR:]
    q = aggf / jnp.maximum(cnt, 1.0) + h_ref[...] * rootf_ref[...]
    qh, ql = _split(q)
    f = fh_ref[...]
    out_ref[...] = _bdot(qh, f) + _bdot(ql, f) + biasf_ref[...]


_final_call = pl.pallas_call(
    _final_body,
    out_shape=jax.ShapeDtypeStruct((NR, 8), jnp.float32),
)


def kernel(x, edge_index, edge_attr, Wm0, bm0, root0, bias0, gamma0, beta0,
           Wm1, bm1, root1, bias1, gamma1, beta1, Wm2, bm2, root2, bias2,
           gamma2, beta2, Wmf, bmf, rootf, biasf):
    f32 = jnp.float32
    bf16 = jnp.bfloat16
    src = edge_index[0].reshape(IDX_ROWS, CHUNK)
    dst = edge_index[1].reshape(IDX_ROWS, CHUNK)
    zeros_t = jnp.zeros((NPT, D), f32)
    ones_c = jnp.ones((CHUNK, D), f32)
    eye8 = jnp.eye(8, dtype=f32)

    def wsplit(w):
        hi = w.astype(bf16)
        lo = (w - hi.astype(f32)).astype(bf16)
        return hi, lo

    def kron1(w):
        e8 = eye8.astype(w.dtype)
        if w.shape[1] == D * D:
            wt = w.reshape(FT, D, D)
            return jnp.einsum('xy,qio->xqiyo', e8, wt).reshape(128, 16 * 128)
        return jnp.einsum('xy,qo->xqyo', e8, w).reshape(128, 8 * w.shape[1])

    def kron_w(w):
        hi, lo = wsplit(w)
        kh, kl = kron1(hi), kron1(lo)
        return kh, jnp.concatenate([kl, kh], axis=0)

    def bm_cols(bm):
        return jnp.broadcast_to(bm.reshape(D, 1, D), (D, 8, D)).reshape(
            1, 16 * 128
        )

    def tile8v(v):
        return jnp.tile(v.reshape(1, D), (1, 8))

    r8 = jnp.einsum(
        'xy,ki,o->xkiyo', eye8, jnp.eye(D, dtype=f32), jnp.ones((D,), f32)
    ).reshape(128, 16 * 128).astype(bf16)
    r2 = jnp.concatenate([r8, r8], axis=0)
    f8h = jnp.einsum('xy,o->xoy', eye8, jnp.ones((D,), f32)).reshape(
        128, 8
    ).astype(bf16)

    sc_count = _make_sc_count()
    sc_gather = _make_sc_gather()
    sc_scatter = _make_sc_scatter()

    cacc = sc_count(dst, ones_c, zeros_t).reshape(2 * NR, 128)
    ea128 = edge_attr.reshape(ER, 128)

    h = x
    h128 = x.reshape(NR, 128)
    for Wm, bm, root, bias, gamma, beta in (
        (Wm0, bm0, root0, bias0, gamma0, beta0),
        (Wm1, bm1, root1, bias1, gamma1, beta1),
        (Wm2, bm2, root2, bias2, gamma2, beta2),
    ):
        hs = sc_gather(h, src).reshape(ER, 128)
        wh, wl = kron_w(Wm)
        msg = _msg_call(ea128, hs, wh, wl, bm_cols(bm), r2)
        pacc = sc_scatter(msg.reshape(E, D), dst, zeros_t).reshape(2 * NR, 128)
        rh, rl = kron_w(root)
        h128 = _node_call(pacc, cacc, h128, rh, rl, tile8v(bias),
                          tile8v(gamma), tile8v(beta))
        h = h128.reshape(N, D)

    hs = sc_gather(h, src).reshape(ER, 128)
    wfh, wfl = kron_w(Wmf)
    prodf = _msgf_call(ea128, hs, wfh, wfl, tile8v(bmf))
    paccf = sc_scatter(prodf.reshape(E, D), dst, zeros_t).reshape(2 * NR, 128)
    out8 = _final_call(paccf, cacc, h128, tile8v(rootf.reshape(D)),
                       biasf.reshape(1, 1), f8h)
    return out8.reshape(N, 1)
